# baseline (speedup 1.0000x reference)
import functools

import jax
import jax.numpy as jnp
from jax import lax
from jax.experimental import pallas as pl
from jax.experimental.pallas import tpu as pltpu
from jax.experimental.pallas import tpu_sc as plsc

K = 10
NEG = -1000000000.0
ROWS = 128
COLS = 32768
BLOCK_ROWS = 8
SEG = 128
NSEG = COLS // SEG
IDXW = 16
NC = 2
NS = 16
NWORK = NC * NS
PERW = ROWS * IDXW // NWORK


def _tc_body(x_ref, o_ref, idx_ref):
    data = x_ref[...]
    o_ref[...] = data
    r = data.shape[0]
    smax = jnp.max(data, axis=2)
    segiota = lax.broadcasted_iota(jnp.int32, (r, NSEG), 1)
    laneiota = lax.broadcasted_iota(jnp.int32, (r, SEG), 1)
    neginf = jnp.float32(-jnp.inf)
    idxs = []
    for _ in range(K):
        m = jnp.max(smax, axis=1, keepdims=True)
        seg = jnp.min(
            jnp.where(smax == m, segiota, jnp.int32(NSEG)), axis=1, keepdims=True
        )
        rows = []
        for i in range(r):
            sl = x_ref[i, pl.ds(seg[i, 0], 1), :]
            rows.append(sl)
        ext = jnp.concatenate(rows, axis=0)
        gcol = seg * SEG + laneiota
        for prev in idxs:
            ext = jnp.where(gcol == prev, neginf, ext)
        idx = jnp.min(
            jnp.where(ext == m, gcol, jnp.int32(COLS)), axis=1, keepdims=True
        )
        idxs.append(idx)
        newmax = jnp.max(jnp.where(gcol == idx, neginf, ext), axis=1, keepdims=True)
        smax = jnp.where(segiota == seg, newmax, smax)
    flat = jnp.concatenate(idxs + [idxs[0]] * (IDXW - K), axis=1)
    base = pl.program_id(0) * BLOCK_ROWS + lax.broadcasted_iota(
        jnp.int32, (r, IDXW), 0
    )
    idx_ref[...] = flat + base * COLS


_tc_call = pl.pallas_call(
    _tc_body,
    grid=(ROWS // BLOCK_ROWS,),
    in_specs=[pl.BlockSpec((BLOCK_ROWS, NSEG, SEG), lambda i: (i, 0, 0))],
    out_specs=[
        pl.BlockSpec((BLOCK_ROWS, NSEG, SEG), lambda i: (i, 0, 0)),
        pl.BlockSpec((BLOCK_ROWS, IDXW), lambda i: (i, 0)),
    ],
    out_shape=[
        jax.ShapeDtypeStruct((ROWS, NSEG, SEG), jnp.float32),
        jax.ShapeDtypeStruct((ROWS, IDXW), jnp.int32),
    ],
)

@functools.cache
def _get_sc_scatter():
    mesh = plsc.VectorSubcoreMesh(core_axis_name="c", subcore_axis_name="s")

    @functools.partial(
        pl.kernel,
        mesh=mesh,
        scratch_types=[
            pltpu.VMEM((PERW,), jnp.int32),
            pltpu.VMEM((PERW,), jnp.float32),
            pltpu.SemaphoreType.DMA,
        ],
    )
    def _sc_scatter(out_ref, idx_ref, idx_v, vals_v, sem):
        w = lax.axis_index("s") * NC + lax.axis_index("c")
        base = w * PERW
        pltpu.sync_copy(idx_ref.at[pl.ds(base, PERW)], idx_v)
        for c in range(PERW // 16):
            vals_v[pl.ds(c * 16, 16)] = jnp.full((16,), NEG, jnp.float32)
        pltpu.async_copy(vals_v, out_ref.at[idx_v], sem).wait()

    return _sc_scatter


def kernel(x):
    out3, idx = _tc_call(x.reshape(ROWS, NSEG, SEG))
    o_ref = jax.new_ref(out3.reshape(ROWS * COLS))
    _get_sc_scatter()(o_ref, idx.reshape(ROWS * IDXW))
    return o_ref[...].reshape(ROWS, COLS)


# speedup vs baseline: 1.8336x; 1.8336x over previous
import functools

import jax
import jax.numpy as jnp
from jax import lax
from jax.experimental import pallas as pl
from jax.experimental.pallas import tpu as pltpu
from jax.experimental.pallas import tpu_sc as plsc

K = 10
NEG = -1000000000.0
ROWS = 128
COLS = 32768
BLOCK_ROWS = 8
SEG = 128
NSEG = COLS // SEG
IDXW = 16
NC = 2
NS = 16
NWORK = NC * NS
PERW = ROWS * IDXW // NWORK


def _tc_body_single(x_ref, idx_ref):
    data = x_ref[...]
    r = ROWS
    smax = jnp.max(data, axis=2)
    segiota = lax.broadcasted_iota(jnp.int32, (r, NSEG), 1)
    neginf = jnp.float32(-jnp.inf)

    sm = smax
    segs = []
    for _ in range(K):
        m = jnp.max(sm, axis=1, keepdims=True)
        sj = jnp.min(
            jnp.where(sm == m, segiota, jnp.int32(NSEG)), axis=1, keepdims=True
        )
        segs.append(sj)
        sm = jnp.where(segiota == sj, neginf, sm)
    segmat = jnp.concatenate(segs, axis=1)

    siota = lax.broadcasted_iota(jnp.int32, (r, K, NSEG), 2)
    onehot = (siota == segmat.reshape(r, K, 1)).astype(jnp.float32)
    ext = jax.lax.dot_general(
        onehot,
        data,
        (((2,), (1,)), ((0,), (0,))),
        preferred_element_type=jnp.float32,
    ).reshape(r, K * SEG)

    laneiota = lax.broadcasted_iota(jnp.int32, (r, K, SEG), 2)
    gcol = (segmat.reshape(r, K, 1) * SEG + laneiota).reshape(r, K * SEG)
    idxs = []
    for _ in range(K):
        m = jnp.max(ext, axis=1, keepdims=True)
        idx = jnp.min(
            jnp.where(ext == m, gcol, jnp.int32(COLS)), axis=1, keepdims=True
        )
        idxs.append(idx)
        ext = jnp.where(gcol == idx, neginf, ext)
    flat = jnp.concatenate(idxs + [idxs[0]] * (IDXW - K), axis=1)
    base = lax.broadcasted_iota(jnp.int32, (r, IDXW), 0)
    idx_ref[...] = flat + base * COLS


def _tc_body(x_ref, idx_ref):
    data = x_ref[...]
    r = data.shape[0]
    data2 = data.reshape(r * NSEG, SEG)
    smax = jnp.max(data, axis=2)
    segiota = lax.broadcasted_iota(jnp.int32, (r, NSEG), 1)
    neginf = jnp.float32(-jnp.inf)
    sm = smax
    segs = []
    for _ in range(K):
        m = jnp.max(sm, axis=1, keepdims=True)
        sj = jnp.min(
            jnp.where(sm == m, segiota, jnp.int32(NSEG)), axis=1, keepdims=True
        )
        segs.append(sj)
        sm = jnp.where(segiota == sj, neginf, sm)
    segmat = jnp.concatenate(segs, axis=1)

    riota = lax.broadcasted_iota(jnp.int32, (r, K, 1), 0)
    qiota = lax.broadcasted_iota(jnp.int32, (r, K, r * NSEG), 2)
    target = riota * NSEG + segmat.reshape(r, K, 1)
    onehot = (qiota == target).astype(jnp.float32).reshape(r * K, r * NSEG)
    ext = jax.lax.dot_general(
        onehot,
        data2,
        (((1,), (0,)), ((), ())),
        preferred_element_type=jnp.float32,
    ).reshape(r, K * SEG)

    laneiota = lax.broadcasted_iota(jnp.int32, (r, K, SEG), 2)
    gcol = (segmat.reshape(r, K, 1) * SEG + laneiota).reshape(r, K * SEG)
    idxs = []
    for _ in range(K):
        m = jnp.max(ext, axis=1, keepdims=True)
        idx = jnp.min(
            jnp.where(ext == m, gcol, jnp.int32(COLS)), axis=1, keepdims=True
        )
        idxs.append(idx)
        ext = jnp.where(gcol == idx, neginf, ext)
    flat = jnp.concatenate(idxs + [idxs[0]] * (IDXW - K), axis=1)
    base = pl.program_id(0) * BLOCK_ROWS + lax.broadcasted_iota(
        jnp.int32, (r, IDXW), 0
    )
    idx_ref[...] = flat + base * COLS


_tc_call = pl.pallas_call(
    _tc_body_single,
    in_specs=[pl.BlockSpec((ROWS, NSEG, SEG), lambda: (0, 0, 0))],
    out_specs=pl.BlockSpec((ROWS, IDXW), lambda: (0, 0)),
    out_shape=jax.ShapeDtypeStruct((ROWS, IDXW), jnp.int32),
)


@functools.cache
def _get_sc_scatter():
    mesh = plsc.VectorSubcoreMesh(core_axis_name="c", subcore_axis_name="s")

    @functools.partial(
        pl.kernel,
        mesh=mesh,
        scratch_types=[
            pltpu.VMEM((PERW,), jnp.int32),
            pltpu.VMEM((PERW,), jnp.float32),
            pltpu.SemaphoreType.DMA,
        ],
    )
    def _sc_scatter(out_ref, idx_ref, idx_v, vals_v, sem):
        w = lax.axis_index("s") * NC + lax.axis_index("c")
        base = w * PERW
        pltpu.sync_copy(idx_ref.at[pl.ds(base, PERW)], idx_v)
        for c in range(PERW // 16):
            vals_v[pl.ds(c * 16, 16)] = jnp.full((16,), NEG, jnp.float32)
        pltpu.async_copy(vals_v, out_ref.at[idx_v], sem).wait()

    return _sc_scatter


def kernel(x):
    o_ref = jax.new_ref(x.reshape(ROWS * COLS))
    idx = _tc_call(x.reshape(ROWS, NSEG, SEG))
    _get_sc_scatter()(o_ref, idx.reshape(ROWS * IDXW))
    return o_ref[...].reshape(ROWS, COLS)
